# baseline (speedup 1.0000x reference)
import functools

import jax
import jax.numpy as jnp
from jax import lax
from jax.experimental import pallas as pl
from jax.experimental.pallas import tpu as pltpu
from jax.experimental.pallas import tpu_sc as plsc


def _grad_body(d_ref, o_ref):
    d0 = d_ref[0]
    d1 = d_ref[1]
    d2 = d_ref[2]
    dist = jnp.sqrt(d0 * d0 + d1 * d1 + d2 * d2 + 0.25)
    e = jnp.exp(-dist)
    fac = (e * (jnp.cos(dist) - jnp.sin(dist)) + 0.2 * dist) / dist
    o_ref[0] = -d0 * fac
    o_ref[1] = -d1 * fac
    o_ref[2] = -d2 * fac


def _pair_grad(d3):
    M = d3.shape[1]
    EB = 12800
    assert M % EB == 0
    return pl.pallas_call(
        _grad_body,
        out_shape=jax.ShapeDtypeStruct((3, M), jnp.float32),
        grid=(M // EB,),
        in_specs=[pl.BlockSpec((3, EB), lambda i: (0, i))],
        out_specs=pl.BlockSpec((3, EB), lambda i: (0, i)),
    )(d3)


_CHUNK = 4000
_NPAD = 10240
_RB = 64
_BW = _RB * 4
_U = 5


def _make_scatter(B, E, N):
    NC, NS = 2, 16
    assert B % NC == 0
    ept = E // NS
    assert ept % _CHUNK == 0 and _CHUNK % (16 * _U) == 0
    nchunks = ept // _CHUNK
    nvec = _CHUNK // 16
    nb = B // NC
    rpt = _NPAD // NS
    accw = _NPAD * 4
    opt = rpt * 4
    bpt = rpt // _RB
    BE = B * E

    mesh = plsc.VectorSubcoreMesh(core_axis_name="c", subcore_axis_name="s")

    @functools.partial(
        pl.kernel,
        out_type=jax.ShapeDtypeStruct((B * _NPAD * 4,), jnp.float32),
        mesh=mesh,
        compiler_params=pltpu.CompilerParams(
            needs_layout_passes=False, use_tc_tiling_on_sc=False),
        scratch_types=(
            [pltpu.VMEM((2, _CHUNK), jnp.float32) for _ in range(3)]
            + [pltpu.VMEM((2, _CHUNK), jnp.int32),
               pltpu.VMEM((accw,), jnp.float32),
               pltpu.VMEM((opt,), jnp.float32),
               pltpu.VMEM((2 * _BW,), jnp.float32),
               pltpu.VMEM((32,), jnp.int32),
               pltpu.VMEM((NS, 32), jnp.int32),
               pltpu.VMEM_SHARED((NS, accw), jnp.float32),
               pltpu.VMEM_SHARED((NS, 32), jnp.int32),
               pltpu.SemaphoreType.DMA,
               pltpu.SemaphoreType.DMA,
               pltpu.SemaphoreType.DMA,
               pltpu.SemaphoreType.DMA]
        ),
    )
    def scatter_kernel(val_hbm, pairs_hbm, out_hbm,
                       vx2, vy2, vz2, idx2, acc_v, obuf, tmp2,
                       metab, metall, partials, meta_sh,
                       ssem0, ssem1, psem, rsem):
        cid = lax.axis_index("c")
        sid = lax.axis_index("s")
        ssems = (ssem0, ssem1)

        zvec = jnp.zeros((16,), jnp.float32)

        for j in range(nb):
            b = j * NC + cid
            base = b * E + sid * ept

            def fire(ci):
                p = ci & 1
                g0 = base + ci * _CHUNK
                s = ssems[p]
                return [
                    pltpu.async_copy(val_hbm.at[pl.ds(g0, _CHUNK)],
                                     vx2.at[p], s),
                    pltpu.async_copy(val_hbm.at[pl.ds(BE + g0, _CHUNK)],
                                     vy2.at[p], s),
                    pltpu.async_copy(val_hbm.at[pl.ds(2 * BE + g0, _CHUNK)],
                                     vz2.at[p], s),
                    pltpu.async_copy(pairs_hbm.at[pl.ds(g0, _CHUNK)],
                                     idx2.at[p], s),
                ]

            hs = fire(0)

            pltpu.sync_copy(pairs_hbm.at[pl.ds(base, 16)],
                            metab.at[pl.ds(0, 16)])
            pltpu.sync_copy(pairs_hbm.at[pl.ds(base + ept - 16, 16)],
                            metab.at[pl.ds(16, 16)])
            lo = lax.reduce_min(metab[pl.ds(0, 16)], axes=(0,))
            hi = lax.reduce_max(metab[pl.ds(16, 16)], axes=(0,))
            blk0 = lo // _RB
            nblk = (hi + _RB) // _RB - blk0
            w0 = blk0 * _BW

            with jax.named_scope("zero_acc"):
                def zbody(i, carry):
                    acc_v[pl.ds(w0 + i * 16, 16)] = zvec
                    return carry
                lax.fori_loop(0, nblk * (_BW // 16), zbody, 0)

            with jax.named_scope("edges"):
                for ci in range(nchunks):
                    p = ci & 1
                    for h in hs:
                        h.wait()
                    if ci + 1 < nchunks:
                        hs = fire(ci + 1)
                    vxp, vyp, vzp, idxp = (vx2.at[p], vy2.at[p], vz2.at[p],
                                           idx2.at[p])

                    def body(i, carry):
                        for u in range(_U):
                            sl = pl.ds((i * _U + u) * 16, 16)
                            a0 = idxp[sl] * 4
                            plsc.addupdate_scatter(acc_v, [a0], vxp[sl])
                            plsc.addupdate_scatter(acc_v, [a0 + 1], vyp[sl])
                            plsc.addupdate_scatter(acc_v, [a0 + 2], vzp[sl])
                        return carry
                    lax.fori_loop(0, nvec // _U, body, 0)

            with jax.named_scope("publish"):
                def pbody(k, carry):
                    o = w0 + k * _BW
                    pltpu.async_copy(acc_v.at[pl.ds(o, _BW)],
                                     partials.at[sid, pl.ds(o, _BW)], psem)
                    return carry
                lax.fori_loop(0, nblk, pbody, 0)
                metab[pl.ds(0, 16)] = jnp.full((16,), 1, jnp.int32) * blk0
                metab[pl.ds(16, 16)] = jnp.full((16,), 1, jnp.int32) * nblk
                pltpu.sync_copy(metab, meta_sh.at[sid])

                def dbody(k, carry):
                    pltpu.make_async_copy(
                        acc_v.at[pl.ds(0, _BW)],
                        partials.at[sid, pl.ds(0, _BW)], psem).wait()
                    return carry
                lax.fori_loop(0, nblk, dbody, 0)
            with jax.named_scope("barrier1"):
                plsc.subcore_barrier()

            ns_owner = jax.named_scope("owner")
            ns_owner.__enter__()

            def obody(i, carry):
                obuf[pl.ds(i * 16, 16)] = zvec
                return carry
            lax.fori_loop(0, opt // 16, obody, 0)
            pltpu.sync_copy(meta_sh, metall)
            myblk0 = sid * bpt
            for t in range(NS):
                tb0 = lax.reduce_max(metall[t, pl.ds(0, 16)], axes=(0,))
                tnb = lax.reduce_max(metall[t, pl.ds(16, 16)], axes=(0,))
                ov0 = lax.max(tb0, myblk0)
                cnt = lax.max(lax.min(tb0 + tnb, myblk0 + bpt) - ov0, 0)

                @pl.when(cnt > 0)
                def _():
                    pltpu.async_copy(
                        partials.at[t, pl.ds(ov0 * _BW, _BW)],
                        tmp2.at[pl.ds(0, _BW)], rsem)

                def rbody(k, carry):
                    pltpu.make_async_copy(
                        partials.at[t, pl.ds(0, _BW)],
                        tmp2.at[pl.ds(0, _BW)], rsem).wait()

                    @pl.when(k + 1 < cnt)
                    def _():
                        pltpu.async_copy(
                            partials.at[t, pl.ds((ov0 + k + 1) * _BW, _BW)],
                            tmp2.at[pl.ds(((k + 1) & 1) * _BW, _BW)], rsem)
                    oo = (ov0 + k - myblk0) * _BW
                    tb = (k & 1) * _BW
                    for i in range(_BW // 16):
                        s = pl.ds(oo + i * 16, 16)
                        obuf[s] = obuf[s] + tmp2[pl.ds(tb + i * 16, 16)]
                    return carry
                lax.fori_loop(0, cnt, rbody, 0)

            pltpu.sync_copy(
                obuf, out_hbm.at[pl.ds(b * accw + sid * opt, opt)])
            ns_owner.__exit__(None, None, None)
            with jax.named_scope("barrier2"):
                plsc.subcore_barrier()

    return scatter_kernel


def kernel(diff, pairs, R):
    B, E, D = diff.shape
    N = R.shape[1]
    assert D == 3
    NS = 16
    S = E // NS // 16

    def _perm(a, lead):
        return (a.reshape(lead + (NS, 16, S))
                .swapaxes(len(lead) + 1, len(lead) + 2)
                .reshape(lead + (E,)))

    d3 = jnp.moveaxis(diff, -1, 0).reshape(3, B, E)
    d3 = _perm(d3, (3, B)).reshape(3, B * E)
    val_flat = _pair_grad(d3).reshape(3 * B * E)

    pairs_flat = _perm(pairs.astype(jnp.int32), (B,)).reshape(B * E)

    out_flat = _make_scatter(B, E, N)(val_flat, pairs_flat)
    out4 = out_flat.reshape(B, _NPAD, 4)
    return out4[:, :N, :3]


# speedup vs baseline: 1.5739x; 1.5739x over previous
import functools

import jax
import jax.numpy as jnp
from jax import lax
from jax.experimental import pallas as pl
from jax.experimental.pallas import tpu as pltpu
from jax.experimental.pallas import tpu_sc as plsc


def _grad_body(d_ref, o_ref):
    d0 = d_ref[0]
    d1 = d_ref[1]
    d2 = d_ref[2]
    dist = jnp.sqrt(d0 * d0 + d1 * d1 + d2 * d2 + 0.25)
    e = jnp.exp(-dist)
    fac = (e * (jnp.cos(dist) - jnp.sin(dist)) + 0.2 * dist) / dist
    o_ref[0] = -d0 * fac
    o_ref[1] = -d1 * fac
    o_ref[2] = -d2 * fac


def _pair_grad(d3):
    M = d3.shape[1]
    EB = 12800
    assert M % EB == 0
    return pl.pallas_call(
        _grad_body,
        out_shape=jax.ShapeDtypeStruct((3, M), jnp.float32),
        grid=(M // EB,),
        in_specs=[pl.BlockSpec((3, EB), lambda i: (0, i))],
        out_specs=pl.BlockSpec((3, EB), lambda i: (0, i)),
    )(d3)


_CHUNK = 4000
_NPAD = 10240
_RB = 64
_BW = _RB * 4
_U = 5


def _make_scatter(B, E, N):
    NC, NS = 2, 16
    assert B % NC == 0
    ept = E // NS
    assert ept % _CHUNK == 0 and _CHUNK % (16 * _U) == 0
    nchunks = ept // _CHUNK
    nvec = _CHUNK // 16
    nb = B // NC
    rpt = _NPAD // NS
    accw = _NPAD * 4
    opt = rpt * 4
    bpt = rpt // _RB
    BE = B * E

    mesh = plsc.VectorSubcoreMesh(core_axis_name="c", subcore_axis_name="s")

    @functools.partial(
        pl.kernel,
        out_type=jax.ShapeDtypeStruct((B * _NPAD * 4,), jnp.float32),
        mesh=mesh,
        compiler_params=pltpu.CompilerParams(
            needs_layout_passes=False, use_tc_tiling_on_sc=False),
        scratch_types=(
            [pltpu.VMEM((2, _CHUNK), jnp.float32) for _ in range(3)]
            + [pltpu.VMEM((2, _CHUNK), jnp.int32),
               pltpu.VMEM((accw,), jnp.float32),
               pltpu.VMEM((opt,), jnp.float32),
               pltpu.VMEM((2 * _BW,), jnp.float32),
               pltpu.VMEM((32,), jnp.int32),
               pltpu.VMEM((NS, 32), jnp.int32),
               pltpu.VMEM_SHARED((NS, accw), jnp.float32),
               pltpu.VMEM_SHARED((NS, 32), jnp.int32),
               pltpu.SemaphoreType.DMA,
               pltpu.SemaphoreType.DMA,
               pltpu.SemaphoreType.DMA,
               pltpu.SemaphoreType.DMA]
        ),
    )
    def scatter_kernel(val_hbm, pairs_hbm, out_hbm,
                       vx2, vy2, vz2, idx2, acc_v, obuf, tmp2,
                       metab, metall, partials, meta_sh,
                       ssem0, ssem1, psem, rsem):
        cid = lax.axis_index("c")
        sid = lax.axis_index("s")
        ssems = (ssem0, ssem1)

        zvec = jnp.zeros((16,), jnp.float32)

        for j in range(nb):
            b = j * NC + cid
            base = b * E + sid * ept

            def fire(ci):
                p = ci & 1
                g0 = base + ci * _CHUNK
                s = ssems[p]
                return [
                    pltpu.async_copy(val_hbm.at[pl.ds(g0, _CHUNK)],
                                     vx2.at[p], s),
                    pltpu.async_copy(val_hbm.at[pl.ds(BE + g0, _CHUNK)],
                                     vy2.at[p], s),
                    pltpu.async_copy(val_hbm.at[pl.ds(2 * BE + g0, _CHUNK)],
                                     vz2.at[p], s),
                    pltpu.async_copy(pairs_hbm.at[pl.ds(g0, _CHUNK)],
                                     idx2.at[p], s),
                ]

            hs = fire(0)

            pltpu.sync_copy(pairs_hbm.at[pl.ds(base, 16)],
                            metab.at[pl.ds(0, 16)])
            pltpu.sync_copy(pairs_hbm.at[pl.ds(base + ept - 16, 16)],
                            metab.at[pl.ds(16, 16)])
            lo = lax.reduce_min(metab[pl.ds(0, 16)], axes=(0,))
            hi = lax.reduce_max(metab[pl.ds(16, 16)], axes=(0,))
            blk0 = lo // _RB
            nblk = (hi + _RB) // _RB - blk0
            w0 = blk0 * _BW

            with jax.named_scope("zero_acc"):
                def zbody(i, carry):
                    acc_v[pl.ds(w0 + i * 16, 16)] = zvec
                    return carry
                lax.fori_loop(0, nblk * (_BW // 16), zbody, 0)

            with jax.named_scope("edges"):
                for ci in range(nchunks):
                    p = ci & 1
                    for h in hs:
                        h.wait()
                    if ci + 1 < nchunks:
                        hs = fire(ci + 1)
                    vxp, vyp, vzp, idxp = (vx2.at[p], vy2.at[p], vz2.at[p],
                                           idx2.at[p])

                    lane_off = lax.iota(jnp.int32, 16) * nvec

                    def body(i, carry):
                        for u in range(_U):
                            pos = lane_off + (i * _U + u)
                            a0 = plsc.load_gather(idxp, [pos]) * 4
                            plsc.addupdate_scatter(
                                acc_v, [a0], plsc.load_gather(vxp, [pos]))
                            plsc.addupdate_scatter(
                                acc_v, [a0 + 1], plsc.load_gather(vyp, [pos]))
                            plsc.addupdate_scatter(
                                acc_v, [a0 + 2], plsc.load_gather(vzp, [pos]))
                        return carry
                    lax.fori_loop(0, nvec // _U, body, 0)

            with jax.named_scope("publish"):
                def pbody(k, carry):
                    o = w0 + k * _BW
                    pltpu.async_copy(acc_v.at[pl.ds(o, _BW)],
                                     partials.at[sid, pl.ds(o, _BW)], psem)
                    return carry
                lax.fori_loop(0, nblk, pbody, 0)
                metab[pl.ds(0, 16)] = jnp.full((16,), 1, jnp.int32) * blk0
                metab[pl.ds(16, 16)] = jnp.full((16,), 1, jnp.int32) * nblk
                pltpu.sync_copy(metab, meta_sh.at[sid])

                def dbody(k, carry):
                    pltpu.make_async_copy(
                        acc_v.at[pl.ds(0, _BW)],
                        partials.at[sid, pl.ds(0, _BW)], psem).wait()
                    return carry
                lax.fori_loop(0, nblk, dbody, 0)
            with jax.named_scope("barrier1"):
                plsc.subcore_barrier()

            ns_owner = jax.named_scope("owner")
            ns_owner.__enter__()

            def obody(i, carry):
                obuf[pl.ds(i * 16, 16)] = zvec
                return carry
            lax.fori_loop(0, opt // 16, obody, 0)
            pltpu.sync_copy(meta_sh, metall)
            myblk0 = sid * bpt
            for t in range(NS):
                tb0 = lax.reduce_max(metall[t, pl.ds(0, 16)], axes=(0,))
                tnb = lax.reduce_max(metall[t, pl.ds(16, 16)], axes=(0,))
                ov0 = lax.max(tb0, myblk0)
                cnt = lax.max(lax.min(tb0 + tnb, myblk0 + bpt) - ov0, 0)

                @pl.when(cnt > 0)
                def _():
                    pltpu.async_copy(
                        partials.at[t, pl.ds(ov0 * _BW, _BW)],
                        tmp2.at[pl.ds(0, _BW)], rsem)

                def rbody(k, carry):
                    pltpu.make_async_copy(
                        partials.at[t, pl.ds(0, _BW)],
                        tmp2.at[pl.ds(0, _BW)], rsem).wait()

                    @pl.when(k + 1 < cnt)
                    def _():
                        pltpu.async_copy(
                            partials.at[t, pl.ds((ov0 + k + 1) * _BW, _BW)],
                            tmp2.at[pl.ds(((k + 1) & 1) * _BW, _BW)], rsem)
                    oo = (ov0 + k - myblk0) * _BW
                    tb = (k & 1) * _BW
                    for i in range(_BW // 16):
                        s = pl.ds(oo + i * 16, 16)
                        obuf[s] = obuf[s] + tmp2[pl.ds(tb + i * 16, 16)]
                    return carry
                lax.fori_loop(0, cnt, rbody, 0)

            pltpu.sync_copy(
                obuf, out_hbm.at[pl.ds(b * accw + sid * opt, opt)])
            ns_owner.__exit__(None, None, None)
            with jax.named_scope("barrier2"):
                plsc.subcore_barrier()

    return scatter_kernel


def kernel(diff, pairs, R):
    B, E, D = diff.shape
    N = R.shape[1]
    assert D == 3
    d3 = jnp.moveaxis(diff, -1, 0).reshape(3, B * E)
    val_flat = _pair_grad(d3).reshape(3 * B * E)

    pairs_flat = pairs.astype(jnp.int32).reshape(B * E)

    out_flat = _make_scatter(B, E, N)(val_flat, pairs_flat)
    out4 = out_flat.reshape(B, _NPAD, 4)
    return out4[:, :N, :3]


# speedup vs baseline: 1.9900x; 1.2644x over previous
import functools

import jax
import jax.numpy as jnp
from jax import lax
from jax.experimental import pallas as pl
from jax.experimental.pallas import tpu as pltpu
from jax.experimental.pallas import tpu_sc as plsc


def _grad_body(d_ref, o_ref):
    d0 = d_ref[0]
    d1 = d_ref[1]
    d2 = d_ref[2]
    dist = jnp.sqrt(d0 * d0 + d1 * d1 + d2 * d2 + 0.25)
    e = jnp.exp(-dist)
    fac = (e * (jnp.cos(dist) - jnp.sin(dist)) + 0.2 * dist) / dist
    o_ref[0] = -d0 * fac
    o_ref[1] = -d1 * fac
    o_ref[2] = -d2 * fac


def _pair_grad(d3):
    M = d3.shape[1]
    LN = 1280
    SB = 40
    R = M // LN
    assert M % LN == 0 and R % SB == 0
    out = pl.pallas_call(
        _grad_body,
        out_shape=jax.ShapeDtypeStruct((3, R, LN), jnp.float32),
        grid=(R // SB,),
        in_specs=[pl.BlockSpec((3, SB, LN), lambda i: (0, i, 0))],
        out_specs=pl.BlockSpec((3, SB, LN), lambda i: (0, i, 0)),
        compiler_params=pltpu.CompilerParams(
            allow_input_fusion=[True]),
    )(d3.reshape(3, R, LN))
    return out.reshape(3, M)


_CHUNK = 4000
_NPAD = 10240
_RB = 64
_BW = _RB * 4
_U = 5


def _make_scatter(B, E, N):
    NC, NS = 2, 16
    assert B % NC == 0
    ept = E // NS
    assert ept % _CHUNK == 0 and _CHUNK % (16 * _U) == 0
    nchunks = ept // _CHUNK
    nvec = _CHUNK // 16
    nb = B // NC
    rpt = _NPAD // NS
    accw = _NPAD * 4
    opt = rpt * 4
    bpt = rpt // _RB
    BE = B * E

    mesh = plsc.VectorSubcoreMesh(core_axis_name="c", subcore_axis_name="s")

    @functools.partial(
        pl.kernel,
        out_type=jax.ShapeDtypeStruct((B * _NPAD * 4,), jnp.float32),
        mesh=mesh,
        compiler_params=pltpu.CompilerParams(
            needs_layout_passes=False, use_tc_tiling_on_sc=False),
        scratch_types=(
            [pltpu.VMEM((2, _CHUNK), jnp.float32) for _ in range(3)]
            + [pltpu.VMEM((2, _CHUNK), jnp.int32),
               pltpu.VMEM((accw,), jnp.float32),
               pltpu.VMEM((opt,), jnp.float32),
               pltpu.VMEM((2 * _BW,), jnp.float32),
               pltpu.VMEM((32,), jnp.int32),
               pltpu.VMEM((NS, 32), jnp.int32),
               pltpu.VMEM_SHARED((NS, accw), jnp.float32),
               pltpu.VMEM_SHARED((NS, 32), jnp.int32),
               pltpu.SemaphoreType.DMA,
               pltpu.SemaphoreType.DMA,
               pltpu.SemaphoreType.DMA,
               pltpu.SemaphoreType.DMA]
        ),
    )
    def scatter_kernel(val_hbm, pairs_hbm, out_hbm,
                       vx2, vy2, vz2, idx2, acc_v, obuf, tmp2,
                       metab, metall, partials, meta_sh,
                       ssem0, ssem1, psem, rsem):
        cid = lax.axis_index("c")
        sid = lax.axis_index("s")
        ssems = (ssem0, ssem1)

        zvec = jnp.zeros((16,), jnp.float32)

        for j in range(nb):
            b = j * NC + cid
            base = b * E + sid * ept

            def fire(ci):
                p = ci & 1
                g0 = base + ci * _CHUNK
                s = ssems[p]
                return [
                    pltpu.async_copy(val_hbm.at[pl.ds(g0, _CHUNK)],
                                     vx2.at[p], s),
                    pltpu.async_copy(val_hbm.at[pl.ds(BE + g0, _CHUNK)],
                                     vy2.at[p], s),
                    pltpu.async_copy(val_hbm.at[pl.ds(2 * BE + g0, _CHUNK)],
                                     vz2.at[p], s),
                    pltpu.async_copy(pairs_hbm.at[pl.ds(g0, _CHUNK)],
                                     idx2.at[p], s),
                ]

            hs = fire(0)

            pltpu.sync_copy(pairs_hbm.at[pl.ds(base, 16)],
                            metab.at[pl.ds(0, 16)])
            pltpu.sync_copy(pairs_hbm.at[pl.ds(base + ept - 16, 16)],
                            metab.at[pl.ds(16, 16)])
            lo = lax.reduce_min(metab[pl.ds(0, 16)], axes=(0,))
            hi = lax.reduce_max(metab[pl.ds(16, 16)], axes=(0,))
            blk0 = lo // _RB
            nblk = (hi + _RB) // _RB - blk0
            w0 = blk0 * _BW

            with jax.named_scope("zero_acc"):
                def zbody(i, carry):
                    acc_v[pl.ds(w0 + i * 16, 16)] = zvec
                    return carry
                lax.fori_loop(0, nblk * (_BW // 16), zbody, 0)

            with jax.named_scope("edges"):
                for ci in range(nchunks):
                    p = ci & 1
                    for h in hs:
                        h.wait()
                    if ci + 1 < nchunks:
                        hs = fire(ci + 1)
                    vxp, vyp, vzp, idxp = (vx2.at[p], vy2.at[p], vz2.at[p],
                                           idx2.at[p])

                    lane_off = lax.iota(jnp.int32, 16) * nvec

                    def body(i, carry):
                        for u in range(_U):
                            pos = lane_off + (i * _U + u)
                            a0 = plsc.load_gather(idxp, [pos]) * 4
                            plsc.addupdate_scatter(
                                acc_v, [a0], plsc.load_gather(vxp, [pos]))
                            plsc.addupdate_scatter(
                                acc_v, [a0 + 1], plsc.load_gather(vyp, [pos]))
                            plsc.addupdate_scatter(
                                acc_v, [a0 + 2], plsc.load_gather(vzp, [pos]))
                        return carry
                    lax.fori_loop(0, nvec // _U, body, 0)

            with jax.named_scope("publish"):
                def pbody(k, carry):
                    o = w0 + k * _BW
                    pltpu.async_copy(acc_v.at[pl.ds(o, _BW)],
                                     partials.at[sid, pl.ds(o, _BW)], psem)
                    return carry
                lax.fori_loop(0, nblk, pbody, 0)
                metab[pl.ds(0, 16)] = jnp.full((16,), 1, jnp.int32) * blk0
                metab[pl.ds(16, 16)] = jnp.full((16,), 1, jnp.int32) * nblk
                pltpu.sync_copy(metab, meta_sh.at[sid])

                def dbody(k, carry):
                    pltpu.make_async_copy(
                        acc_v.at[pl.ds(0, _BW)],
                        partials.at[sid, pl.ds(0, _BW)], psem).wait()
                    return carry
                lax.fori_loop(0, nblk, dbody, 0)
            with jax.named_scope("barrier1"):
                plsc.subcore_barrier()

            ns_owner = jax.named_scope("owner")
            ns_owner.__enter__()

            def obody(i, carry):
                obuf[pl.ds(i * 16, 16)] = zvec
                return carry
            lax.fori_loop(0, opt // 16, obody, 0)
            pltpu.sync_copy(meta_sh, metall)
            myblk0 = sid * bpt
            for t in range(NS):
                tb0 = lax.reduce_max(metall[t, pl.ds(0, 16)], axes=(0,))
                tnb = lax.reduce_max(metall[t, pl.ds(16, 16)], axes=(0,))
                ov0 = lax.max(tb0, myblk0)
                cnt = lax.max(lax.min(tb0 + tnb, myblk0 + bpt) - ov0, 0)

                @pl.when(cnt > 0)
                def _():
                    pltpu.async_copy(
                        partials.at[t, pl.ds(ov0 * _BW, _BW)],
                        tmp2.at[pl.ds(0, _BW)], rsem)

                def rbody(k, carry):
                    pltpu.make_async_copy(
                        partials.at[t, pl.ds(0, _BW)],
                        tmp2.at[pl.ds(0, _BW)], rsem).wait()

                    @pl.when(k + 1 < cnt)
                    def _():
                        pltpu.async_copy(
                            partials.at[t, pl.ds((ov0 + k + 1) * _BW, _BW)],
                            tmp2.at[pl.ds(((k + 1) & 1) * _BW, _BW)], rsem)
                    oo = (ov0 + k - myblk0) * _BW
                    tb = (k & 1) * _BW
                    for i in range(_BW // 16):
                        s = pl.ds(oo + i * 16, 16)
                        obuf[s] = obuf[s] + tmp2[pl.ds(tb + i * 16, 16)]
                    return carry
                lax.fori_loop(0, cnt, rbody, 0)

            pltpu.sync_copy(
                obuf, out_hbm.at[pl.ds(b * accw + sid * opt, opt)])
            ns_owner.__exit__(None, None, None)
            with jax.named_scope("barrier2"):
                plsc.subcore_barrier()

    return scatter_kernel


def kernel(diff, pairs, R):
    B, E, D = diff.shape
    N = R.shape[1]
    assert D == 3
    d3 = jnp.moveaxis(diff, -1, 0).reshape(3, B * E)
    val_flat = _pair_grad(d3).reshape(3 * B * E)

    pairs_flat = pairs.astype(jnp.int32).reshape(B * E)

    out_flat = _make_scatter(B, E, N)(val_flat, pairs_flat)
    out4 = out_flat.reshape(B, _NPAD, 4)
    return out4[:, :N, :3]


# speedup vs baseline: 2.1495x; 1.0802x over previous
import functools

import jax
import jax.numpy as jnp
from jax import lax
from jax.experimental import pallas as pl
from jax.experimental.pallas import tpu as pltpu
from jax.experimental.pallas import tpu_sc as plsc


def _grad_body(d_ref, o_ref):
    d0 = d_ref[0]
    d1 = d_ref[1]
    d2 = d_ref[2]
    dist = jnp.sqrt(d0 * d0 + d1 * d1 + d2 * d2 + 0.25)
    e = jnp.exp(-dist)
    fac = (e * (jnp.cos(dist) - jnp.sin(dist)) + 0.2 * dist) / dist
    o_ref[0] = -d0 * fac
    o_ref[1] = -d1 * fac
    o_ref[2] = -d2 * fac


def _pair_grad(d3):
    M = d3.shape[1]
    LN = 1280
    SB = 40
    R = M // LN
    assert M % LN == 0 and R % SB == 0
    out = pl.pallas_call(
        _grad_body,
        out_shape=jax.ShapeDtypeStruct((3, R, LN), jnp.float32),
        grid=(R // SB,),
        in_specs=[pl.BlockSpec((3, SB, LN), lambda i: (0, i, 0))],
        out_specs=pl.BlockSpec((3, SB, LN), lambda i: (0, i, 0)),
        compiler_params=pltpu.CompilerParams(
            allow_input_fusion=[True]),
    )(d3.reshape(3, R, LN))
    return out.reshape(3, M)


_CHUNK = 4000
_NPAD = 10240
_RB = 64
_BW = _RB * 4
_U = 10


def _make_scatter(B, E, N):
    NC, NS = 2, 16
    assert B % NC == 0
    ept = E // NS
    assert ept % _CHUNK == 0 and _CHUNK % (16 * _U) == 0
    nchunks = ept // _CHUNK
    nvec = _CHUNK // 16
    nb = B // NC
    rpt = _NPAD // NS
    accw = _NPAD * 4
    opt = rpt * 4
    bpt = rpt // _RB
    BE = B * E

    mesh = plsc.VectorSubcoreMesh(core_axis_name="c", subcore_axis_name="s")

    @functools.partial(
        pl.kernel,
        out_type=jax.ShapeDtypeStruct((B * N * 3,), jnp.float32),
        mesh=mesh,
        compiler_params=pltpu.CompilerParams(
            needs_layout_passes=False, use_tc_tiling_on_sc=False),
        scratch_types=(
            [pltpu.VMEM((2, _CHUNK), jnp.float32) for _ in range(3)]
            + [pltpu.VMEM((2, _CHUNK), jnp.int32),
               pltpu.VMEM((accw,), jnp.float32),
               pltpu.VMEM((opt,), jnp.float32),
               pltpu.VMEM((2 * _BW,), jnp.float32),
               pltpu.VMEM((32,), jnp.int32),
               pltpu.VMEM((NS, 32), jnp.int32),
               pltpu.VMEM_SHARED((NS, accw), jnp.float32),
               pltpu.VMEM_SHARED((NS, 32), jnp.int32),
               pltpu.SemaphoreType.DMA,
               pltpu.SemaphoreType.DMA,
               pltpu.SemaphoreType.DMA,
               pltpu.SemaphoreType.DMA]
        ),
    )
    def scatter_kernel(val_hbm, pairs_hbm, out_hbm,
                       vx2, vy2, vz2, idx2, acc_v, obuf, tmp2,
                       metab, metall, partials, meta_sh,
                       ssem0, ssem1, psem, rsem):
        cid = lax.axis_index("c")
        sid = lax.axis_index("s")
        ssems = (ssem0, ssem1)

        zvec = jnp.zeros((16,), jnp.float32)

        for j in range(nb):
            b = j * NC + cid
            base = b * E + sid * ept

            def fire(ci):
                p = ci & 1
                g0 = base + ci * _CHUNK
                s = ssems[p]
                return [
                    pltpu.async_copy(val_hbm.at[pl.ds(g0, _CHUNK)],
                                     vx2.at[p], s),
                    pltpu.async_copy(val_hbm.at[pl.ds(BE + g0, _CHUNK)],
                                     vy2.at[p], s),
                    pltpu.async_copy(val_hbm.at[pl.ds(2 * BE + g0, _CHUNK)],
                                     vz2.at[p], s),
                    pltpu.async_copy(pairs_hbm.at[pl.ds(g0, _CHUNK)],
                                     idx2.at[p], s),
                ]

            hs = fire(0)

            pltpu.sync_copy(pairs_hbm.at[pl.ds(base, 16)],
                            metab.at[pl.ds(0, 16)])
            pltpu.sync_copy(pairs_hbm.at[pl.ds(base + ept - 16, 16)],
                            metab.at[pl.ds(16, 16)])
            lo = lax.reduce_min(metab[pl.ds(0, 16)], axes=(0,))
            hi = lax.reduce_max(metab[pl.ds(16, 16)], axes=(0,))
            blk0 = lo // _RB
            nblk = (hi + _RB) // _RB - blk0
            w0 = blk0 * _BW

            with jax.named_scope("zero_acc"):
                def zbody(i, carry):
                    acc_v[pl.ds(w0 + i * 16, 16)] = zvec
                    return carry
                lax.fori_loop(0, nblk * (_BW // 16), zbody, 0)

            with jax.named_scope("edges"):
                for ci in range(nchunks):
                    p = ci & 1
                    for h in hs:
                        h.wait()
                    if ci + 1 < nchunks:
                        hs = fire(ci + 1)
                    vxp, vyp, vzp, idxp = (vx2.at[p], vy2.at[p], vz2.at[p],
                                           idx2.at[p])

                    lane_off = lax.iota(jnp.int32, 16) * nvec

                    def body(i, carry):
                        for u in range(_U):
                            pos = lane_off + (i * _U + u)
                            a0 = plsc.load_gather(idxp, [pos]) * 4
                            plsc.addupdate_scatter(
                                acc_v, [a0], plsc.load_gather(vxp, [pos]))
                            plsc.addupdate_scatter(
                                acc_v, [a0 + 1], plsc.load_gather(vyp, [pos]))
                            plsc.addupdate_scatter(
                                acc_v, [a0 + 2], plsc.load_gather(vzp, [pos]))
                        return carry
                    lax.fori_loop(0, nvec // _U, body, 0)

            with jax.named_scope("publish"):
                def pbody(k, carry):
                    o = w0 + k * _BW
                    pltpu.async_copy(acc_v.at[pl.ds(o, _BW)],
                                     partials.at[sid, pl.ds(o, _BW)], psem)
                    return carry
                lax.fori_loop(0, nblk, pbody, 0)
                metab[pl.ds(0, 16)] = jnp.full((16,), 1, jnp.int32) * blk0
                metab[pl.ds(16, 16)] = jnp.full((16,), 1, jnp.int32) * nblk
                pltpu.sync_copy(metab, meta_sh.at[sid])

                def dbody(k, carry):
                    pltpu.make_async_copy(
                        acc_v.at[pl.ds(0, _BW)],
                        partials.at[sid, pl.ds(0, _BW)], psem).wait()
                    return carry
                lax.fori_loop(0, nblk, dbody, 0)
            with jax.named_scope("barrier1"):
                plsc.subcore_barrier()

            ns_owner = jax.named_scope("owner")
            ns_owner.__enter__()

            def obody(i, carry):
                obuf[pl.ds(i * 16, 16)] = zvec
                return carry
            lax.fori_loop(0, opt // 16, obody, 0)
            pltpu.sync_copy(meta_sh, metall)
            myblk0 = sid * bpt
            for t in range(NS):
                tb0 = lax.reduce_max(metall[t, pl.ds(0, 16)], axes=(0,))
                tnb = lax.reduce_max(metall[t, pl.ds(16, 16)], axes=(0,))
                ov0 = lax.max(tb0, myblk0)
                cnt = lax.max(lax.min(tb0 + tnb, myblk0 + bpt) - ov0, 0)

                @pl.when(cnt > 0)
                def _():
                    pltpu.async_copy(
                        partials.at[t, pl.ds(ov0 * _BW, _BW)],
                        tmp2.at[pl.ds(0, _BW)], rsem)

                def rbody(k, carry):
                    pltpu.make_async_copy(
                        partials.at[t, pl.ds(0, _BW)],
                        tmp2.at[pl.ds(0, _BW)], rsem).wait()

                    @pl.when(k + 1 < cnt)
                    def _():
                        pltpu.async_copy(
                            partials.at[t, pl.ds((ov0 + k + 1) * _BW, _BW)],
                            tmp2.at[pl.ds(((k + 1) & 1) * _BW, _BW)], rsem)
                    oo = (ov0 + k - myblk0) * _BW
                    tb = (k & 1) * _BW
                    for i in range(_BW // 16):
                        s = pl.ds(oo + i * 16, 16)
                        obuf[s] = obuf[s] + tmp2[pl.ds(tb + i * 16, 16)]
                    return carry
                lax.fori_loop(0, cnt, rbody, 0)

            iota = lax.iota(jnp.int32, 16)
            npk = rpt * 3

            def kbody(g, carry):
                pos = g * 16 + iota
                src = (pos // 3) * 4 + pos % 3
                v = plsc.load_gather(obuf, [src])
                obuf[pl.ds(g * 16, 16)] = v
                return carry
            lax.fori_loop(0, npk // 16, kbody, 0)

            dst0 = b * (N * 3) + sid * npk
            last = N * 3 - 15 * npk

            @pl.when(sid < NS - 1)
            def _():
                pltpu.sync_copy(obuf.at[pl.ds(0, npk)],
                                out_hbm.at[pl.ds(dst0, npk)])

            @pl.when(sid == NS - 1)
            def _():
                pltpu.sync_copy(obuf.at[pl.ds(0, last)],
                                out_hbm.at[pl.ds(dst0, last)])
            ns_owner.__exit__(None, None, None)
            with jax.named_scope("barrier2"):
                plsc.subcore_barrier()

    return scatter_kernel


def kernel(diff, pairs, R):
    B, E, D = diff.shape
    N = R.shape[1]
    assert D == 3
    d3 = jnp.moveaxis(diff, -1, 0).reshape(3, B * E)
    val_flat = _pair_grad(d3).reshape(3 * B * E)

    pairs_flat = pairs.astype(jnp.int32).reshape(B * E)

    out_flat = _make_scatter(B, E, N)(val_flat, pairs_flat)
    return out_flat.reshape(B, N, 3)


# speedup vs baseline: 2.3010x; 1.0704x over previous
import functools

import jax
import jax.numpy as jnp
from jax import lax
from jax.experimental import pallas as pl
from jax.experimental.pallas import tpu as pltpu
from jax.experimental.pallas import tpu_sc as plsc


def _grad_body(d_ref, o_ref):
    d0 = d_ref[0]
    d1 = d_ref[1]
    d2 = d_ref[2]
    dist = jnp.sqrt(d0 * d0 + d1 * d1 + d2 * d2 + 0.25)
    e = jnp.exp(-dist)
    fac = (e * (jnp.cos(dist) - jnp.sin(dist)) + 0.2 * dist) / dist
    o_ref[0] = -d0 * fac
    o_ref[1] = -d1 * fac
    o_ref[2] = -d2 * fac


def _pair_grad(d3):
    M = d3.shape[1]
    LN = 1280
    SB = 40
    R = M // LN
    assert M % LN == 0 and R % SB == 0
    out = pl.pallas_call(
        _grad_body,
        out_shape=jax.ShapeDtypeStruct((3, R, LN), jnp.float32),
        grid=(R // SB,),
        in_specs=[pl.BlockSpec((3, SB, LN), lambda i: (0, i, 0))],
        out_specs=pl.BlockSpec((3, SB, LN), lambda i: (0, i, 0)),
        compiler_params=pltpu.CompilerParams(
            allow_input_fusion=[True]),
    )(d3.reshape(3, R, LN))
    return out.reshape(3, M)


_CHUNK = 4000
_NPAD = 10240
_RB = 64
_BW = _RB * 4
_U = 10


def _make_scatter(B, E, N):
    NC, NS = 2, 16
    assert B % NC == 0
    ept = E // NS
    assert ept % _CHUNK == 0 and _CHUNK % (16 * _U) == 0
    nchunks = ept // _CHUNK
    nvec = _CHUNK // 16
    nb = B // NC
    rpt = _NPAD // NS
    accw = _NPAD * 4
    opt = rpt * 4
    bpt = rpt // _RB
    BE = B * E

    mesh = plsc.VectorSubcoreMesh(core_axis_name="c", subcore_axis_name="s")

    @functools.partial(
        pl.kernel,
        out_type=jax.ShapeDtypeStruct((B * N * 3,), jnp.float32),
        mesh=mesh,
        compiler_params=pltpu.CompilerParams(
            needs_layout_passes=False, use_tc_tiling_on_sc=False),
        scratch_types=(
            [pltpu.VMEM((2, _CHUNK), jnp.float32) for _ in range(3)]
            + [pltpu.VMEM((2, _CHUNK), jnp.int32),
               pltpu.VMEM((accw,), jnp.float32),
               pltpu.VMEM((opt,), jnp.float32),
               pltpu.VMEM((2 * _BW,), jnp.float32),
               pltpu.VMEM((32,), jnp.int32),
               pltpu.VMEM((NS, 32), jnp.int32),
               pltpu.VMEM_SHARED((NS, accw), jnp.float32),
               pltpu.VMEM_SHARED((NS, 32), jnp.int32),
               pltpu.SemaphoreType.DMA,
               pltpu.SemaphoreType.DMA,
               pltpu.SemaphoreType.DMA,
               pltpu.SemaphoreType.DMA]
        ),
    )
    def scatter_kernel(val_hbm, pairs_hbm, out_hbm,
                       vx2, vy2, vz2, idx2, acc_v, obuf, tmp2,
                       metab, metall, partials, meta_sh,
                       ssem0, ssem1, psem, rsem):
        cid = lax.axis_index("c")
        sid = lax.axis_index("s")
        ssems = (ssem0, ssem1)

        zvec = jnp.zeros((16,), jnp.float32)

        for j in range(nb):
            b = j * NC + cid
            base = b * E + sid * ept

            def fire(ci):
                p = ci & 1
                g0 = base + ci * _CHUNK
                s = ssems[p]
                return [
                    pltpu.async_copy(val_hbm.at[pl.ds(g0, _CHUNK)],
                                     vx2.at[p], s),
                    pltpu.async_copy(val_hbm.at[pl.ds(BE + g0, _CHUNK)],
                                     vy2.at[p], s),
                    pltpu.async_copy(val_hbm.at[pl.ds(2 * BE + g0, _CHUNK)],
                                     vz2.at[p], s),
                    pltpu.async_copy(pairs_hbm.at[pl.ds(g0, _CHUNK)],
                                     idx2.at[p], s),
                ]

            hs = fire(0)

            pltpu.sync_copy(pairs_hbm.at[pl.ds(base, 16)],
                            metab.at[pl.ds(0, 16)])
            pltpu.sync_copy(pairs_hbm.at[pl.ds(base + ept - 16, 16)],
                            metab.at[pl.ds(16, 16)])
            lo = lax.reduce_min(metab[pl.ds(0, 16)], axes=(0,))
            hi = lax.reduce_max(metab[pl.ds(16, 16)], axes=(0,))
            blk0 = lo // _RB
            nblk = (hi + _RB) // _RB - blk0
            w0 = blk0 * _BW

            with jax.named_scope("zero_acc"):
                def zbody(i, carry):
                    acc_v[pl.ds(w0 + i * 16, 16)] = zvec
                    return carry
                lax.fori_loop(0, nblk * (_BW // 16), zbody, 0)

            with jax.named_scope("edges"):
                for ci in range(nchunks):
                    p = ci & 1
                    for h in hs:
                        h.wait()
                    if ci + 1 < nchunks:
                        hs = fire(ci + 1)
                    vxp, vyp, vzp, idxp = (vx2.at[p], vy2.at[p], vz2.at[p],
                                           idx2.at[p])

                    halfn = _CHUNK // 2
                    halfv = halfn // 16
                    lane_off = lax.iota(jnp.int32, 16) * halfv

                    def body(i, carry):
                        for u in range(_U // 2):
                            k = i * (_U // 2) + u
                            for h in (0, halfn):
                                pos = lane_off + (h + k)
                                a0 = plsc.load_gather(idxp, [pos]) * 4
                                plsc.addupdate_scatter(
                                    acc_v, [a0],
                                    plsc.load_gather(vxp, [pos]))
                                plsc.addupdate_scatter(
                                    acc_v, [a0 + 1],
                                    plsc.load_gather(vyp, [pos]))
                                plsc.addupdate_scatter(
                                    acc_v, [a0 + 2],
                                    plsc.load_gather(vzp, [pos]))
                        return carry
                    lax.fori_loop(0, halfv // (_U // 2), body, 0)

            with jax.named_scope("publish"):
                def pbody(k, carry):
                    o = w0 + k * _BW
                    pltpu.async_copy(acc_v.at[pl.ds(o, _BW)],
                                     partials.at[sid, pl.ds(o, _BW)], psem)
                    return carry
                lax.fori_loop(0, nblk, pbody, 0)
                metab[pl.ds(0, 16)] = jnp.full((16,), 1, jnp.int32) * blk0
                metab[pl.ds(16, 16)] = jnp.full((16,), 1, jnp.int32) * nblk
                pltpu.sync_copy(metab, meta_sh.at[sid])

                def dbody(k, carry):
                    pltpu.make_async_copy(
                        acc_v.at[pl.ds(0, _BW)],
                        partials.at[sid, pl.ds(0, _BW)], psem).wait()
                    return carry
                lax.fori_loop(0, nblk, dbody, 0)
            with jax.named_scope("barrier1"):
                plsc.subcore_barrier()

            ns_owner = jax.named_scope("owner")
            ns_owner.__enter__()

            def obody(i, carry):
                obuf[pl.ds(i * 16, 16)] = zvec
                return carry
            lax.fori_loop(0, opt // 16, obody, 0)
            pltpu.sync_copy(meta_sh, metall)
            myblk0 = sid * bpt
            for t in range(NS):
                tb0 = lax.reduce_max(metall[t, pl.ds(0, 16)], axes=(0,))
                tnb = lax.reduce_max(metall[t, pl.ds(16, 16)], axes=(0,))
                ov0 = lax.max(tb0, myblk0)
                cnt = lax.max(lax.min(tb0 + tnb, myblk0 + bpt) - ov0, 0)

                @pl.when(cnt > 0)
                def _():
                    pltpu.async_copy(
                        partials.at[t, pl.ds(ov0 * _BW, _BW)],
                        tmp2.at[pl.ds(0, _BW)], rsem)

                def rbody(k, carry):
                    pltpu.make_async_copy(
                        partials.at[t, pl.ds(0, _BW)],
                        tmp2.at[pl.ds(0, _BW)], rsem).wait()

                    @pl.when(k + 1 < cnt)
                    def _():
                        pltpu.async_copy(
                            partials.at[t, pl.ds((ov0 + k + 1) * _BW, _BW)],
                            tmp2.at[pl.ds(((k + 1) & 1) * _BW, _BW)], rsem)
                    oo = (ov0 + k - myblk0) * _BW
                    tb = (k & 1) * _BW
                    for i in range(_BW // 16):
                        s = pl.ds(oo + i * 16, 16)
                        obuf[s] = obuf[s] + tmp2[pl.ds(tb + i * 16, 16)]
                    return carry
                lax.fori_loop(0, cnt, rbody, 0)

            iota = lax.iota(jnp.int32, 16)
            npk = rpt * 3

            def kbody(g, carry):
                pos = g * 16 + iota
                src = (pos // 3) * 4 + pos % 3
                v = plsc.load_gather(obuf, [src])
                obuf[pl.ds(g * 16, 16)] = v
                return carry
            lax.fori_loop(0, npk // 16, kbody, 0)

            dst0 = b * (N * 3) + sid * npk
            last = N * 3 - 15 * npk

            @pl.when(sid < NS - 1)
            def _():
                pltpu.sync_copy(obuf.at[pl.ds(0, npk)],
                                out_hbm.at[pl.ds(dst0, npk)])

            @pl.when(sid == NS - 1)
            def _():
                pltpu.sync_copy(obuf.at[pl.ds(0, last)],
                                out_hbm.at[pl.ds(dst0, last)])
            ns_owner.__exit__(None, None, None)
            with jax.named_scope("barrier2"):
                plsc.subcore_barrier()

    return scatter_kernel


def kernel(diff, pairs, R):
    B, E, D = diff.shape
    N = R.shape[1]
    assert D == 3
    d3 = jnp.moveaxis(diff, -1, 0).reshape(3, B * E)
    val_flat = _pair_grad(d3).reshape(3 * B * E)

    pairs_flat = pairs.astype(jnp.int32).reshape(B * E)

    out_flat = _make_scatter(B, E, N)(val_flat, pairs_flat)
    return out_flat.reshape(B, N, 3)
